# baseline (speedup 1.0000x reference)
import jax
import jax.numpy as jnp
from jax import lax
from jax.experimental import pallas as pl
from jax.experimental.pallas import tpu as pltpu
from jax.experimental.pallas import tpu_sc as plsc

_N = 10000
_E = 320000
_D = 128
_NW = 32
_EPW = 10240
_EPAD = _EPW * _NW
_NT = 10080
_R = 1680
_C = 32
_NPAD = 10240


def _proj0_body(x_ref, wk, wq, wv, ws, b, ko, qo, vo, so):
    h = x_ref[...]
    ko[...] = jnp.dot(h, wk[...], preferred_element_type=jnp.float32,
                    precision=lax.Precision.HIGHEST)
    qo[...] = jnp.dot(h, wq[...], preferred_element_type=jnp.float32,
                    precision=lax.Precision.HIGHEST)
    vo[...] = jnp.dot(h, wv[...], preferred_element_type=jnp.float32,
                    precision=lax.Precision.HIGHEST)
    so[...] = jnp.dot(h, ws[...], preferred_element_type=jnp.float32,
                    precision=lax.Precision.HIGHEST) + b[...]


def _proj_mid_body(parts, sprev, wk, wq, wv, ws, b, ko, qo, vo, so):
    h = jnp.maximum(parts[0] + parts[1] + sprev[...], 0.0)
    ko[...] = jnp.dot(h, wk[...], preferred_element_type=jnp.float32,
                    precision=lax.Precision.HIGHEST)
    qo[...] = jnp.dot(h, wq[...], preferred_element_type=jnp.float32,
                    precision=lax.Precision.HIGHEST)
    vo[...] = jnp.dot(h, wv[...], preferred_element_type=jnp.float32,
                    precision=lax.Precision.HIGHEST)
    so[...] = jnp.dot(h, ws[...], preferred_element_type=jnp.float32,
                    precision=lax.Precision.HIGHEST) + b[...]


def _proj_last_body(parts, sprev, wk, wq, wv, ws, ko, qo, vo, so):
    h = jnp.maximum(parts[0] + parts[1] + sprev[...], 0.0)
    ko[...] = jnp.sum(h * wk[...], axis=1, keepdims=True)
    qo[...] = jnp.sum(h * wq[...], axis=1, keepdims=True)
    vo[...] = jnp.sum(h * wv[...], axis=1, keepdims=True)
    so[...] = jnp.sum(h * ws[...], axis=1, keepdims=True)


def _full(shape):
    return pl.BlockSpec(shape, lambda i: tuple(0 for _ in shape))


def _proj0(x, wk, wq, wv, ws, b):
    grid = (_NT // _R,)
    row = pl.BlockSpec((_R, _D), lambda i: (i, 0))
    return pl.pallas_call(
        _proj0_body,
        grid=grid,
        in_specs=[row, _full((_D, _D)), _full((_D, _D)), _full((_D, _D)),
                  _full((_D, _D)), _full((1, _D))],
        out_specs=[row, row, row, row],
        out_shape=[jax.ShapeDtypeStruct((_NT, _D), jnp.float32)] * 4,
    )(x, wk, wq, wv, ws, b.reshape(1, _D))


def _proj_mid(parts, sprev, wk, wq, wv, ws, b):
    grid = (_NT // _R,)
    row = pl.BlockSpec((_R, _D), lambda i: (i, 0))
    prow = pl.BlockSpec((2, _R, _D), lambda i: (0, i, 0))
    return pl.pallas_call(
        _proj_mid_body,
        grid=grid,
        in_specs=[prow, row, _full((_D, _D)), _full((_D, _D)), _full((_D, _D)),
                  _full((_D, _D)), _full((1, _D))],
        out_specs=[row, row, row, row],
        out_shape=[jax.ShapeDtypeStruct((_NT, _D), jnp.float32)] * 4,
    )(parts, sprev, wk, wq, wv, ws, b.reshape(1, _D))


def _proj_last(parts, sprev, wk, wq, wv, ws):
    grid = (_NT // _R,)
    row = pl.BlockSpec((_R, _D), lambda i: (i, 0))
    prow = pl.BlockSpec((2, _R, _D), lambda i: (0, i, 0))
    orow = pl.BlockSpec((_R, 1), lambda i: (i, 0))
    return pl.pallas_call(
        _proj_last_body,
        grid=grid,
        in_specs=[prow, row, _full((1, _D)), _full((1, _D)), _full((1, _D)),
                  _full((1, _D))],
        out_specs=[orow, orow, orow, orow],
        out_shape=[jax.ShapeDtypeStruct((_NT, 1), jnp.float32)] * 4,
    )(parts, sprev, wk.reshape(1, _D), wq.reshape(1, _D),
      wv.reshape(1, _D), ws.reshape(1, _D))


def _edge_kernel(k, q, v, src, dst, d):
    rows_pt = _NPAD // 16
    nchunks = _EPW // _C
    blk = 4 * _C
    mesh = plsc.VectorSubcoreMesh(core_axis_name="c", subcore_axis_name="s",
                                  num_cores=2, num_subcores=16)
    if d:
        zeros = jnp.zeros((_NPAD, d), jnp.float32)
        buf_shape = (_C, d)
        out_t = jax.ShapeDtypeStruct((2, _NPAD, d), jnp.float32)
        shared_t = pltpu.VMEM_SHARED((_NPAD, d), jnp.float32)
        nvec = d // 16
    else:
        zeros = jnp.zeros((_NPAD,), jnp.float32)
        buf_shape = (_C,)
        out_t = jax.ShapeDtypeStruct((2, _NPAD), jnp.float32)
        shared_t = pltpu.VMEM_SHARED((_NPAD,), jnp.float32)
        nvec = _C // 16

    def body(k_hbm, q_hbm, v_hbm, src_hbm, dst_hbm, z_hbm, out_hbm,
             sb0, sb1, db0, db1, dc0, dc1, kb0, kb1, qb0, qb1, vb0, vb1,
             mb0, mb1, shared,
             s00, s01, s02, s10, s11, s12, sc0, sc1):
        sib = (sb0, sb1)
        dib = (db0, db1)
        dcur = (dc0, dc1)
        kb = (kb0, kb1)
        qb = (qb0, qb1)
        vb = (vb0, vb1)
        mb = (mb0, mb1)
        sg = ((s00, s01, s02), (s10, s11, s12))
        ssc = (sc0, sc1)
        cid = lax.axis_index("c")
        sid = lax.axis_index("s")
        wid = sid * 2 + cid
        r0 = pl.multiple_of(sid * rows_pt, 8)
        pltpu.sync_copy(z_hbm.at[pl.ds(r0, rows_pt)],
                        shared.at[pl.ds(r0, rows_pt)])
        plsc.subcore_barrier()
        base_w = wid * _EPW

        def load_blk(b, pb):
            base = base_w + b * blk
            pltpu.sync_copy(src_hbm.at[pl.ds(base, blk)], sib[pb])
            pltpu.sync_copy(dst_hbm.at[pl.ds(base, blk)], dib[pb])

        def gather_start(p, pb, row):
            o = row * _C
            pltpu.async_copy(k_hbm.at[dib[pb].at[pl.ds(o, _C)]], kb[p],
                             sg[p][0])
            pltpu.async_copy(q_hbm.at[sib[pb].at[pl.ds(o, _C)]], qb[p],
                             sg[p][1])
            pltpu.async_copy(v_hbm.at[sib[pb].at[pl.ds(o, _C)]], vb[p],
                             sg[p][2])

        def gather_wait(p, pb, row):
            o = row * _C
            pltpu.make_async_copy(k_hbm.at[dib[pb].at[pl.ds(o, _C)]], kb[p],
                                  sg[p][0]).wait()
            pltpu.make_async_copy(q_hbm.at[sib[pb].at[pl.ds(o, _C)]], qb[p],
                                  sg[p][1]).wait()
            pltpu.make_async_copy(v_hbm.at[sib[pb].at[pl.ds(o, _C)]], vb[p],
                                  sg[p][2]).wait()

        def scatter_wait(p):
            pltpu.make_async_copy(mb[p], shared.at[dcur[p]], ssc[p]).wait()

        def finish(g, p, pb, row):
            @pl.when(g >= 2)
            def _():
                scatter_wait(p)

            gather_wait(p, pb, row)
            for t in range(_C // 16):
                dcur[p][pl.ds(16 * t, 16)] = \
                    dib[pb][pl.ds(row * _C + 16 * t, 16)]
            if d:
                def edge(e, inner):
                    for j in range(nvec):
                        sl = pl.ds(j * 16, 16)
                        gt = 1.0 / (1.0 + jnp.exp(-(kb[p][e, sl]
                                                    + qb[p][e, sl])))
                        mb[p][e, sl] = gt * vb[p][e, sl]
                    return inner

                lax.fori_loop(0, _C, edge, 0)
            else:
                for i in range(nvec):
                    sl = pl.ds(i * 16, 16)
                    gt = 1.0 / (1.0 + jnp.exp(-(kb[p][sl] + qb[p][sl])))
                    mb[p][sl] = gt * vb[p][sl]
            pltpu.async_copy(mb[p], shared.at[dcur[p]], ssc[p], add=True)

        fire_map = [(0, 2), (0, 3), (1, 0), (1, 1), (1, 2), (1, 3),
                    (0, 0), (0, 1)]
        nblocks = nchunks // 4
        load_blk(0, 0)
        load_blk(1, 1)
        gather_start(0, 0, 0)
        gather_start(1, 0, 1)

        def octet(o, carry):
            for cc in range(8):
                g = 8 * o + cc
                p = cc % 2
                finish(g, p, (cc // 4) % 2, cc % 4)
                fb, fr = fire_map[cc]

                @pl.when(g + 2 < nchunks)
                def _():
                    gather_start(p, fb, fr)

                if cc == 3:
                    b2 = 2 * o + 2

                    @pl.when(b2 < nblocks)
                    def _():
                        load_blk(b2, 0)
                if cc == 7:
                    b3 = 2 * o + 3

                    @pl.when(b3 < nblocks)
                    def _():
                        load_blk(b3, 1)
            return carry

        lax.fori_loop(0, nchunks // 8, octet, 0)
        scatter_wait(0)
        scatter_wait(1)

        plsc.subcore_barrier()
        pltpu.sync_copy(shared.at[pl.ds(r0, rows_pt)],
                        out_hbm.at[cid, pl.ds(r0, rows_pt)])

    fn = pl.kernel(
        body,
        out_type=out_t,
        mesh=mesh,
        scratch_types=[
            pltpu.VMEM((blk,), jnp.int32),
            pltpu.VMEM((blk,), jnp.int32),
            pltpu.VMEM((blk,), jnp.int32),
            pltpu.VMEM((blk,), jnp.int32),
            pltpu.VMEM((_C,), jnp.int32),
            pltpu.VMEM((_C,), jnp.int32),
            pltpu.VMEM(buf_shape, jnp.float32),
            pltpu.VMEM(buf_shape, jnp.float32),
            pltpu.VMEM(buf_shape, jnp.float32),
            pltpu.VMEM(buf_shape, jnp.float32),
            pltpu.VMEM(buf_shape, jnp.float32),
            pltpu.VMEM(buf_shape, jnp.float32),
            pltpu.VMEM(buf_shape, jnp.float32),
            pltpu.VMEM(buf_shape, jnp.float32),
            shared_t,
            pltpu.SemaphoreType.DMA,
            pltpu.SemaphoreType.DMA,
            pltpu.SemaphoreType.DMA,
            pltpu.SemaphoreType.DMA,
            pltpu.SemaphoreType.DMA,
            pltpu.SemaphoreType.DMA,
            pltpu.SemaphoreType.DMA,
            pltpu.SemaphoreType.DMA,
        ],
    )
    return fn(k, q, v, src, dst, zeros)


def kernel(x, edge_index, W0k, W0q, W0v, W0s, b0,
           W1k, W1q, W1v, W1s, b1, W2k, W2q, W2v, W2s, b2):
    pad = jnp.full((_EPAD - _E,), _N, dtype=jnp.int32)
    src = jnp.concatenate([edge_index[0], pad])
    dst = jnp.concatenate([edge_index[1], pad])
    xp = jnp.pad(x, ((0, _NT - _N), (0, 0)))

    k0, q0, v0, s0 = _proj0(xp, W0k, W0q, W0v, W0s, b0)
    p0 = _edge_kernel(k0, q0, v0, src, dst, _D)

    k1, q1, v1, s1 = _proj_mid(p0, s0, W1k, W1q, W1v, W1s, b1)
    p1 = _edge_kernel(k1, q1, v1, src, dst, _D)

    k2, q2, v2, s2 = _proj_last(p1, s1, W2k, W2q, W2v, W2s)
    p2 = _edge_kernel(k2.reshape(_NT), q2.reshape(_NT), v2.reshape(_NT),
                      src, dst, 0)

    return (p2[0, :_N] + p2[1, :_N])[:, None] + s2[:_N] + b2


# speedup vs baseline: 1.1059x; 1.1059x over previous
import jax
import jax.numpy as jnp
from jax import lax
from jax.experimental import pallas as pl
from jax.experimental.pallas import tpu as pltpu
from jax.experimental.pallas import tpu_sc as plsc

_N = 10000
_E = 320000
_D = 128
_NW = 32
_EPW = 10368
_EPAD = _EPW * _NW
_NT = 10080
_R = 1680
_C = 48
_NPAD = 10240


def _proj0_body(x_ref, wk, wq, wv, ws, b, ko, qo, vo, so):
    h = x_ref[...]
    ko[...] = jnp.dot(h, wk[...], preferred_element_type=jnp.float32,
                    precision=lax.Precision.HIGHEST)
    qo[...] = jnp.dot(h, wq[...], preferred_element_type=jnp.float32,
                    precision=lax.Precision.HIGHEST)
    vo[...] = jnp.dot(h, wv[...], preferred_element_type=jnp.float32,
                    precision=lax.Precision.HIGHEST)
    so[...] = jnp.dot(h, ws[...], preferred_element_type=jnp.float32,
                    precision=lax.Precision.HIGHEST) + b[...]


def _proj_mid_body(parts, sprev, wk, wq, wv, ws, b, ko, qo, vo, so):
    h = jnp.maximum(parts[0] + parts[1] + sprev[...], 0.0)
    ko[...] = jnp.dot(h, wk[...], preferred_element_type=jnp.float32,
                    precision=lax.Precision.HIGHEST)
    qo[...] = jnp.dot(h, wq[...], preferred_element_type=jnp.float32,
                    precision=lax.Precision.HIGHEST)
    vo[...] = jnp.dot(h, wv[...], preferred_element_type=jnp.float32,
                    precision=lax.Precision.HIGHEST)
    so[...] = jnp.dot(h, ws[...], preferred_element_type=jnp.float32,
                    precision=lax.Precision.HIGHEST) + b[...]


def _proj_last_body(parts, sprev, wk, wq, wv, ws, ko, qo, vo, so):
    h = jnp.maximum(parts[0] + parts[1] + sprev[...], 0.0)
    ko[...] = jnp.sum(h * wk[...], axis=1, keepdims=True)
    qo[...] = jnp.sum(h * wq[...], axis=1, keepdims=True)
    vo[...] = jnp.sum(h * wv[...], axis=1, keepdims=True)
    so[...] = jnp.sum(h * ws[...], axis=1, keepdims=True)


def _full(shape):
    return pl.BlockSpec(shape, lambda i: tuple(0 for _ in shape))


def _proj0(x, wk, wq, wv, ws, b):
    grid = (_NT // _R,)
    row = pl.BlockSpec((_R, _D), lambda i: (i, 0))
    return pl.pallas_call(
        _proj0_body,
        grid=grid,
        in_specs=[row, _full((_D, _D)), _full((_D, _D)), _full((_D, _D)),
                  _full((_D, _D)), _full((1, _D))],
        out_specs=[row, row, row, row],
        out_shape=[jax.ShapeDtypeStruct((_NT, _D), jnp.float32)] * 4,
    )(x, wk, wq, wv, ws, b.reshape(1, _D))


def _proj_mid(parts, sprev, wk, wq, wv, ws, b):
    grid = (_NT // _R,)
    row = pl.BlockSpec((_R, _D), lambda i: (i, 0))
    prow = pl.BlockSpec((2, _R, _D), lambda i: (0, i, 0))
    return pl.pallas_call(
        _proj_mid_body,
        grid=grid,
        in_specs=[prow, row, _full((_D, _D)), _full((_D, _D)), _full((_D, _D)),
                  _full((_D, _D)), _full((1, _D))],
        out_specs=[row, row, row, row],
        out_shape=[jax.ShapeDtypeStruct((_NT, _D), jnp.float32)] * 4,
    )(parts, sprev, wk, wq, wv, ws, b.reshape(1, _D))


def _proj_last(parts, sprev, wk, wq, wv, ws):
    grid = (_NT // _R,)
    row = pl.BlockSpec((_R, _D), lambda i: (i, 0))
    prow = pl.BlockSpec((2, _R, _D), lambda i: (0, i, 0))
    orow = pl.BlockSpec((_R, 1), lambda i: (i, 0))
    return pl.pallas_call(
        _proj_last_body,
        grid=grid,
        in_specs=[prow, row, _full((1, _D)), _full((1, _D)), _full((1, _D)),
                  _full((1, _D))],
        out_specs=[orow, orow, orow, orow],
        out_shape=[jax.ShapeDtypeStruct((_NT, 1), jnp.float32)] * 4,
    )(parts, sprev, wk.reshape(1, _D), wq.reshape(1, _D),
      wv.reshape(1, _D), ws.reshape(1, _D))


def _edge_schedule(nchunks, load_blk, fire, finish):
    nblocks = nchunks // 4
    load_blk(0, 0)
    load_blk(1, 1)
    fire(0, 0, 0)

    def octet(o, carry):
        for cc in range(8):
            g = 8 * o + cc
            p = cc % 2
            nb_, nr = [(0, 1), (0, 2), (0, 3), (1, 0), (1, 1), (1, 2),
                       (1, 3), (0, 0)][cc]

            @pl.when(g + 1 < nchunks)
            def _():
                fire(1 - p, nb_, nr)

            finish(p)
            if cc == 3:
                b2 = 2 * o + 2

                @pl.when(b2 < nblocks)
                def _():
                    load_blk(b2, 0)
            if cc == 7:
                b3 = 2 * o + 3

                @pl.when(b3 < nblocks)
                def _():
                    load_blk(b3, 1)
        return carry

    lax.fori_loop(0, nchunks // 8, octet, 0)


def _edge_aggr(k, q, v, src, dst):
    rows_pt = _NPAD // 16
    nb = _D // 16
    nchunks = _EPW // _C
    mesh = plsc.VectorSubcoreMesh(core_axis_name="c", subcore_axis_name="s",
                                  num_cores=2, num_subcores=16)
    zeros = jnp.zeros((_NPAD, _D), jnp.float32)

    def body(k_hbm, q_hbm, v_hbm, src_hbm, dst_hbm, z_hbm, out_hbm,
             sb0, sb1, db0, db1, si0, si1, di0, di1,
             kb0, kb1, qb0, qb1, vb0, vb1, shared,
             s00, s01, s02, s10, s11, s12):
        sib = (sb0, sb1)
        dib = (db0, db1)
        si = (si0, si1)
        di = (di0, di1)
        kb = (kb0, kb1)
        qb = (qb0, qb1)
        vb = (vb0, vb1)
        sg = ((s00, s01, s02), (s10, s11, s12))
        cid = lax.axis_index("c")
        sid = lax.axis_index("s")
        wid = sid * 2 + cid
        r0 = pl.multiple_of(sid * rows_pt, 8)
        pltpu.sync_copy(z_hbm.at[pl.ds(r0, rows_pt)],
                        shared.at[pl.ds(r0, rows_pt)])
        plsc.subcore_barrier()
        base_w = wid * _EPW

        gd = [
            (pltpu.make_async_copy(k_hbm.at[di[p]], kb[p], sg[p][0]),
             pltpu.make_async_copy(q_hbm.at[si[p]], qb[p], sg[p][1]),
             pltpu.make_async_copy(v_hbm.at[si[p]], vb[p], sg[p][2]))
            for p in (0, 1)
        ]

        def load_blk(b, pb):
            base = base_w + b * (4 * _C)
            pltpu.sync_copy(src_hbm.at[pl.ds(base, 4 * _C)], sib[pb])
            pltpu.sync_copy(dst_hbm.at[pl.ds(base, 4 * _C)], dib[pb])

        def fire(p, pb, row):
            for t in range(_C // 16):
                sl = pl.ds(16 * t, 16)
                bl = pl.ds(row * _C + 16 * t, 16)
                si[p][sl] = sib[pb][bl]
                di[p][sl] = dib[pb][bl]
            for dsc in gd[p]:
                dsc.start()

        def finish(p):
            for dsc in gd[p]:
                dsc.wait()

            def edge(e, inner):
                for j in range(nb):
                    sl = pl.ds(j * 16, 16)
                    g = 1.0 / (1.0 + jnp.exp(-(kb[p][e, sl] + qb[p][e, sl])))
                    kb[p][e, sl] = g * vb[p][e, sl]
                return inner

            lax.fori_loop(0, _C, edge, 0)
            pltpu.sync_copy(kb[p], shared.at[di[p]], add=True)

        _edge_schedule(nchunks, load_blk, fire, finish)

        plsc.subcore_barrier()
        pltpu.sync_copy(shared.at[pl.ds(r0, rows_pt)],
                        out_hbm.at[cid, pl.ds(r0, rows_pt)])

    fn = pl.kernel(
        body,
        out_type=jax.ShapeDtypeStruct((2, _NPAD, _D), jnp.float32),
        mesh=mesh,
        scratch_types=[
            pltpu.VMEM((4 * _C,), jnp.int32),
            pltpu.VMEM((4 * _C,), jnp.int32),
            pltpu.VMEM((4 * _C,), jnp.int32),
            pltpu.VMEM((4 * _C,), jnp.int32),
            pltpu.VMEM((_C,), jnp.int32),
            pltpu.VMEM((_C,), jnp.int32),
            pltpu.VMEM((_C,), jnp.int32),
            pltpu.VMEM((_C,), jnp.int32),
            pltpu.VMEM((_C, _D), jnp.float32),
            pltpu.VMEM((_C, _D), jnp.float32),
            pltpu.VMEM((_C, _D), jnp.float32),
            pltpu.VMEM((_C, _D), jnp.float32),
            pltpu.VMEM((_C, _D), jnp.float32),
            pltpu.VMEM((_C, _D), jnp.float32),
            pltpu.VMEM_SHARED((_NPAD, _D), jnp.float32),
            pltpu.SemaphoreType.DMA,
            pltpu.SemaphoreType.DMA,
            pltpu.SemaphoreType.DMA,
            pltpu.SemaphoreType.DMA,
            pltpu.SemaphoreType.DMA,
            pltpu.SemaphoreType.DMA,
        ],
    )
    return fn(k, q, v, src, dst, zeros)


def _edge_aggr_1d(k, q, v, src, dst):
    rows_pt = _NPAD // 16
    nchunks = _EPW // _C
    mesh = plsc.VectorSubcoreMesh(core_axis_name="c", subcore_axis_name="s",
                                  num_cores=2, num_subcores=16)
    zeros = jnp.zeros((_NPAD,), jnp.float32)

    def body(k_hbm, q_hbm, v_hbm, src_hbm, dst_hbm, z_hbm, out_hbm,
             sb0, sb1, db0, db1, si0, si1, di0, di1,
             kb0, kb1, qb0, qb1, vb0, vb1, shared,
             s00, s01, s02, s10, s11, s12):
        sib = (sb0, sb1)
        dib = (db0, db1)
        si = (si0, si1)
        di = (di0, di1)
        kb = (kb0, kb1)
        qb = (qb0, qb1)
        vb = (vb0, vb1)
        sg = ((s00, s01, s02), (s10, s11, s12))
        cid = lax.axis_index("c")
        sid = lax.axis_index("s")
        wid = sid * 2 + cid
        r0 = pl.multiple_of(sid * rows_pt, 8)
        pltpu.sync_copy(z_hbm.at[pl.ds(r0, rows_pt)],
                        shared.at[pl.ds(r0, rows_pt)])
        plsc.subcore_barrier()
        base_w = wid * _EPW

        gd = [
            (pltpu.make_async_copy(k_hbm.at[di[p]], kb[p], sg[p][0]),
             pltpu.make_async_copy(q_hbm.at[si[p]], qb[p], sg[p][1]),
             pltpu.make_async_copy(v_hbm.at[si[p]], vb[p], sg[p][2]))
            for p in (0, 1)
        ]

        def load_blk(b, pb):
            base = base_w + b * (4 * _C)
            pltpu.sync_copy(src_hbm.at[pl.ds(base, 4 * _C)], sib[pb])
            pltpu.sync_copy(dst_hbm.at[pl.ds(base, 4 * _C)], dib[pb])

        def fire(p, pb, row):
            for t in range(_C // 16):
                sl = pl.ds(16 * t, 16)
                bl = pl.ds(row * _C + 16 * t, 16)
                si[p][sl] = sib[pb][bl]
                di[p][sl] = dib[pb][bl]
            for dsc in gd[p]:
                dsc.start()

        def finish(p):
            for dsc in gd[p]:
                dsc.wait()

            def vec(i, inner):
                sl = pl.ds(i * 16, 16)
                g = 1.0 / (1.0 + jnp.exp(-(kb[p][sl] + qb[p][sl])))
                kb[p][sl] = g * vb[p][sl]
                return inner

            lax.fori_loop(0, _C // 16, vec, 0)
            pltpu.sync_copy(kb[p], shared.at[di[p]], add=True)

        _edge_schedule(nchunks, load_blk, fire, finish)

        plsc.subcore_barrier()
        pltpu.sync_copy(shared.at[pl.ds(r0, rows_pt)],
                        out_hbm.at[cid, pl.ds(r0, rows_pt)])

    fn = pl.kernel(
        body,
        out_type=jax.ShapeDtypeStruct((2, _NPAD), jnp.float32),
        mesh=mesh,
        scratch_types=[
            pltpu.VMEM((4 * _C,), jnp.int32),
            pltpu.VMEM((4 * _C,), jnp.int32),
            pltpu.VMEM((4 * _C,), jnp.int32),
            pltpu.VMEM((4 * _C,), jnp.int32),
            pltpu.VMEM((_C,), jnp.int32),
            pltpu.VMEM((_C,), jnp.int32),
            pltpu.VMEM((_C,), jnp.int32),
            pltpu.VMEM((_C,), jnp.int32),
            pltpu.VMEM((_C,), jnp.float32),
            pltpu.VMEM((_C,), jnp.float32),
            pltpu.VMEM((_C,), jnp.float32),
            pltpu.VMEM((_C,), jnp.float32),
            pltpu.VMEM((_C,), jnp.float32),
            pltpu.VMEM((_C,), jnp.float32),
            pltpu.VMEM_SHARED((_NPAD,), jnp.float32),
            pltpu.SemaphoreType.DMA,
            pltpu.SemaphoreType.DMA,
            pltpu.SemaphoreType.DMA,
            pltpu.SemaphoreType.DMA,
            pltpu.SemaphoreType.DMA,
            pltpu.SemaphoreType.DMA,
        ],
    )
    return fn(k, q, v, src, dst, zeros)


def kernel(x, edge_index, W0k, W0q, W0v, W0s, b0,
           W1k, W1q, W1v, W1s, b1, W2k, W2q, W2v, W2s, b2):
    pad = jnp.full((_EPAD - _E,), _N, dtype=jnp.int32)
    src = jnp.concatenate([edge_index[0], pad])
    dst = jnp.concatenate([edge_index[1], pad])
    xp = jnp.pad(x, ((0, _NT - _N), (0, 0)))

    k0, q0, v0, s0 = _proj0(xp, W0k, W0q, W0v, W0s, b0)
    p0 = _edge_aggr(k0, q0, v0, src, dst)

    k1, q1, v1, s1 = _proj_mid(p0, s0, W1k, W1q, W1v, W1s, b1)
    p1 = _edge_aggr(k1, q1, v1, src, dst)

    k2, q2, v2, s2 = _proj_last(p1, s1, W2k, W2q, W2v, W2s)
    p2 = _edge_aggr_1d(k2.reshape(_NT), q2.reshape(_NT), v2.reshape(_NT),
                       src, dst)

    return (p2[0, :_N] + p2[1, :_N])[:, None] + s2[:_N] + b2


# speedup vs baseline: 2.0862x; 1.8864x over previous
import jax
import jax.numpy as jnp
from jax import lax
from jax.experimental import pallas as pl
from jax.experimental.pallas import tpu as pltpu
from jax.experimental.pallas import tpu_sc as plsc

_N = 10000
_E = 320000
_D = 128
_NW = 32
_EPW = 10080
_EPAD = _EPW * _NW
_NT = 10080
_R = 1680
_C = 48
_NPAD = 10240


def _proj0_body(x_ref, wk, wq, wv, ws, b, ko, qo, vo, so):
    h = x_ref[...]
    ko[...] = jnp.dot(h, wk[...], preferred_element_type=jnp.float32,
                    precision=lax.Precision.HIGHEST)
    qo[...] = jnp.dot(h, wq[...], preferred_element_type=jnp.float32,
                    precision=lax.Precision.HIGHEST)
    vo[...] = jnp.dot(h, wv[...], preferred_element_type=jnp.float32,
                    precision=lax.Precision.HIGHEST)
    so[...] = jnp.dot(h, ws[...], preferred_element_type=jnp.float32,
                    precision=lax.Precision.HIGHEST) + b[...]


def _proj_mid_body(parts, sprev, wk, wq, wv, ws, b, ko, qo, vo, so):
    h = jnp.maximum(parts[0] + parts[1] + sprev[...], 0.0)
    ko[...] = jnp.dot(h, wk[...], preferred_element_type=jnp.float32,
                    precision=lax.Precision.HIGHEST)
    qo[...] = jnp.dot(h, wq[...], preferred_element_type=jnp.float32,
                    precision=lax.Precision.HIGHEST)
    vo[...] = jnp.dot(h, wv[...], preferred_element_type=jnp.float32,
                    precision=lax.Precision.HIGHEST)
    so[...] = jnp.dot(h, ws[...], preferred_element_type=jnp.float32,
                    precision=lax.Precision.HIGHEST) + b[...]


def _proj_last_body(parts, sprev, wk, wq, wv, ws, ko, qo, vo, so):
    h = jnp.maximum(parts[0] + parts[1] + sprev[...], 0.0)
    ko[...] = jnp.sum(h * wk[...], axis=1, keepdims=True)
    qo[...] = jnp.sum(h * wq[...], axis=1, keepdims=True)
    vo[...] = jnp.sum(h * wv[...], axis=1, keepdims=True)
    so[...] = jnp.sum(h * ws[...], axis=1, keepdims=True)


def _full(shape):
    return pl.BlockSpec(shape, lambda i: tuple(0 for _ in shape))


def _proj0(x, wk, wq, wv, ws, b):
    grid = (_NT // _R,)
    row = pl.BlockSpec((_R, _D), lambda i: (i, 0))
    return pl.pallas_call(
        _proj0_body,
        grid=grid,
        in_specs=[row, _full((_D, _D)), _full((_D, _D)), _full((_D, _D)),
                  _full((_D, _D)), _full((1, _D))],
        out_specs=[row, row, row, row],
        out_shape=[jax.ShapeDtypeStruct((_NT, _D), jnp.float32)] * 4,
    )(x, wk, wq, wv, ws, b.reshape(1, _D))


def _proj_mid(parts, sprev, wk, wq, wv, ws, b):
    grid = (_NT // _R,)
    row = pl.BlockSpec((_R, _D), lambda i: (i, 0))
    prow = pl.BlockSpec((2, _R, _D), lambda i: (0, i, 0))
    return pl.pallas_call(
        _proj_mid_body,
        grid=grid,
        in_specs=[prow, row, _full((_D, _D)), _full((_D, _D)), _full((_D, _D)),
                  _full((_D, _D)), _full((1, _D))],
        out_specs=[row, row, row, row],
        out_shape=[jax.ShapeDtypeStruct((_NT, _D), jnp.float32)] * 4,
    )(parts, sprev, wk, wq, wv, ws, b.reshape(1, _D))


def _proj_last(parts, sprev, wk, wq, wv, ws):
    grid = (_NT // _R,)
    row = pl.BlockSpec((_R, _D), lambda i: (i, 0))
    prow = pl.BlockSpec((2, _R, _D), lambda i: (0, i, 0))
    orow = pl.BlockSpec((_R, 1), lambda i: (i, 0))
    return pl.pallas_call(
        _proj_last_body,
        grid=grid,
        in_specs=[prow, row, _full((1, _D)), _full((1, _D)), _full((1, _D)),
                  _full((1, _D))],
        out_specs=[orow, orow, orow, orow],
        out_shape=[jax.ShapeDtypeStruct((_NT, 1), jnp.float32)] * 4,
    )(parts, sprev, wk.reshape(1, _D), wq.reshape(1, _D),
      wv.reshape(1, _D), ws.reshape(1, _D))


def _edge_pipeline(nchunks, fire, finish):
    fire(0, 0)

    def pair(i, carry):
        for b in (0, 1):
            gg = 2 * i + b
            fire(gg + 1, 1 - b)
            finish(b)
        return carry

    lax.fori_loop(0, (nchunks - 1) // 2, pair, 0)
    for gg in range(2 * ((nchunks - 1) // 2), nchunks - 1):
        fire(gg + 1, (gg + 1) % 2)
        finish(gg % 2)
    finish((nchunks - 1) % 2)


def _edge_aggr(k, q, v, src, dst):
    rows_pt = _NPAD // 16
    nb = _D // 16
    nchunks = _EPW // _C
    mesh = plsc.VectorSubcoreMesh(core_axis_name="c", subcore_axis_name="s",
                                  num_cores=2, num_subcores=16)
    zeros = jnp.zeros((_NPAD, _D), jnp.float32)

    def body(k_hbm, q_hbm, v_hbm, src_hbm, dst_hbm, z_hbm, out_hbm,
             si0, si1, di0, di1, kb0, kb1, qb0, qb1, vb0, vb1, shared,
             s00, s01, s02, s10, s11, s12):
        si = (si0, si1)
        di = (di0, di1)
        kb = (kb0, kb1)
        qb = (qb0, qb1)
        vb = (vb0, vb1)
        sg = ((s00, s01, s02), (s10, s11, s12))
        cid = lax.axis_index("c")
        sid = lax.axis_index("s")
        wid = sid * 2 + cid
        r0 = pl.multiple_of(sid * rows_pt, 8)
        pltpu.sync_copy(z_hbm.at[pl.ds(r0, rows_pt)],
                        shared.at[pl.ds(r0, rows_pt)])
        plsc.subcore_barrier()
        base_w = wid * _EPW

        gd = [
            (pltpu.make_async_copy(k_hbm.at[di[p]], kb[p], sg[p][0]),
             pltpu.make_async_copy(q_hbm.at[si[p]], qb[p], sg[p][1]),
             pltpu.make_async_copy(v_hbm.at[si[p]], vb[p], sg[p][2]))
            for p in (0, 1)
        ]

        def fire(g, p):
            base = base_w + g * _C
            pltpu.sync_copy(src_hbm.at[pl.ds(base, _C)], si[p])
            pltpu.sync_copy(dst_hbm.at[pl.ds(base, _C)], di[p])
            for dsc in gd[p]:
                dsc.start()

        def finish(p):
            for dsc in gd[p]:
                dsc.wait()

            def edge(e, inner):
                for j in range(nb):
                    sl = pl.ds(j * 16, 16)
                    g = 1.0 / (1.0 + jnp.exp(-(kb[p][e, sl] + qb[p][e, sl])))
                    kb[p][e, sl] = g * vb[p][e, sl]
                return inner

            lax.fori_loop(0, _C, edge, 0)
            pltpu.sync_copy(kb[p], shared.at[di[p]], add=True)

        _edge_pipeline(nchunks, fire, finish)

        plsc.subcore_barrier()
        pltpu.sync_copy(shared.at[pl.ds(r0, rows_pt)],
                        out_hbm.at[cid, pl.ds(r0, rows_pt)])

    fn = pl.kernel(
        body,
        out_type=jax.ShapeDtypeStruct((2, _NPAD, _D), jnp.float32),
        mesh=mesh,
        scratch_types=[
            pltpu.VMEM((_C,), jnp.int32),
            pltpu.VMEM((_C,), jnp.int32),
            pltpu.VMEM((_C,), jnp.int32),
            pltpu.VMEM((_C,), jnp.int32),
            pltpu.VMEM((_C, _D), jnp.float32),
            pltpu.VMEM((_C, _D), jnp.float32),
            pltpu.VMEM((_C, _D), jnp.float32),
            pltpu.VMEM((_C, _D), jnp.float32),
            pltpu.VMEM((_C, _D), jnp.float32),
            pltpu.VMEM((_C, _D), jnp.float32),
            pltpu.VMEM_SHARED((_NPAD, _D), jnp.float32),
            pltpu.SemaphoreType.DMA,
            pltpu.SemaphoreType.DMA,
            pltpu.SemaphoreType.DMA,
            pltpu.SemaphoreType.DMA,
            pltpu.SemaphoreType.DMA,
            pltpu.SemaphoreType.DMA,
        ],
    )
    return fn(k, q, v, src, dst, zeros)


def _edge_aggr_1d(k, q, v, src, dst):
    rows_pt = _NPAD // 16
    nchunks = _EPW // _C
    mesh = plsc.VectorSubcoreMesh(core_axis_name="c", subcore_axis_name="s",
                                  num_cores=2, num_subcores=16)
    zeros = jnp.zeros((_NPAD,), jnp.float32)

    def body(k_hbm, q_hbm, v_hbm, src_hbm, dst_hbm, z_hbm, out_hbm,
             si0, si1, di0, di1, kb0, kb1, qb0, qb1, vb0, vb1, shared,
             s00, s01, s02, s10, s11, s12):
        si = (si0, si1)
        di = (di0, di1)
        kb = (kb0, kb1)
        qb = (qb0, qb1)
        vb = (vb0, vb1)
        sg = ((s00, s01, s02), (s10, s11, s12))
        cid = lax.axis_index("c")
        sid = lax.axis_index("s")
        wid = sid * 2 + cid
        r0 = pl.multiple_of(sid * rows_pt, 8)
        pltpu.sync_copy(z_hbm.at[pl.ds(r0, rows_pt)],
                        shared.at[pl.ds(r0, rows_pt)])
        plsc.subcore_barrier()
        base_w = wid * _EPW

        gd = [
            (pltpu.make_async_copy(k_hbm.at[di[p]], kb[p], sg[p][0]),
             pltpu.make_async_copy(q_hbm.at[si[p]], qb[p], sg[p][1]),
             pltpu.make_async_copy(v_hbm.at[si[p]], vb[p], sg[p][2]))
            for p in (0, 1)
        ]

        def fire(g, p):
            base = base_w + g * _C
            pltpu.sync_copy(src_hbm.at[pl.ds(base, _C)], si[p])
            pltpu.sync_copy(dst_hbm.at[pl.ds(base, _C)], di[p])
            for dsc in gd[p]:
                dsc.start()

        def finish(p):
            for dsc in gd[p]:
                dsc.wait()

            def vec(i, inner):
                sl = pl.ds(i * 16, 16)
                g = 1.0 / (1.0 + jnp.exp(-(kb[p][sl] + qb[p][sl])))
                kb[p][sl] = g * vb[p][sl]
                return inner

            lax.fori_loop(0, _C // 16, vec, 0)
            pltpu.sync_copy(kb[p], shared.at[di[p]], add=True)

        _edge_pipeline(nchunks, fire, finish)

        plsc.subcore_barrier()
        pltpu.sync_copy(shared.at[pl.ds(r0, rows_pt)],
                        out_hbm.at[cid, pl.ds(r0, rows_pt)])

    fn = pl.kernel(
        body,
        out_type=jax.ShapeDtypeStruct((2, _NPAD), jnp.float32),
        mesh=mesh,
        scratch_types=[
            pltpu.VMEM((_C,), jnp.int32),
            pltpu.VMEM((_C,), jnp.int32),
            pltpu.VMEM((_C,), jnp.int32),
            pltpu.VMEM((_C,), jnp.int32),
            pltpu.VMEM((_C,), jnp.float32),
            pltpu.VMEM((_C,), jnp.float32),
            pltpu.VMEM((_C,), jnp.float32),
            pltpu.VMEM((_C,), jnp.float32),
            pltpu.VMEM((_C,), jnp.float32),
            pltpu.VMEM((_C,), jnp.float32),
            pltpu.VMEM_SHARED((_NPAD,), jnp.float32),
            pltpu.SemaphoreType.DMA,
            pltpu.SemaphoreType.DMA,
            pltpu.SemaphoreType.DMA,
            pltpu.SemaphoreType.DMA,
            pltpu.SemaphoreType.DMA,
            pltpu.SemaphoreType.DMA,
        ],
    )
    return fn(k, q, v, src, dst, zeros)


def kernel(x, edge_index, W0k, W0q, W0v, W0s, b0,
           W1k, W1q, W1v, W1s, b1, W2k, W2q, W2v, W2s, b2):
    pad = jnp.full((_EPAD - _E,), _N, dtype=jnp.int32)
    src = jnp.concatenate([edge_index[0], pad])
    dst = jnp.concatenate([edge_index[1], pad])
    xp = jnp.pad(x, ((0, _NT - _N), (0, 0)))

    k0, q0, v0, s0 = _proj0(xp, W0k, W0q, W0v, W0s, b0)
    p0 = _edge_aggr(k0, q0, v0, src, dst)

    k1, q1, v1, s1 = _proj_mid(p0, s0, W1k, W1q, W1v, W1s, b1)
    p1 = _edge_aggr(k1, q1, v1, src, dst)

    k2, q2, v2, s2 = _proj_last(p1, s1, W2k, W2q, W2v, W2s)
    p2 = _edge_aggr_1d(k2.reshape(_NT), q2.reshape(_NT), v2.reshape(_NT),
                       src, dst)

    return (p2[0, :_N] + p2[1, :_N])[:, None] + s2[:_N] + b2
